# baseline (speedup 1.0000x reference)
import functools

import jax
import jax.numpy as jnp
from jax import lax
from jax.experimental import pallas as pl
from jax.experimental.pallas import tpu as pltpu
from jax.experimental.pallas import tpu_sc as plsc

BATCH = 16384
VOCAB = 1000000
DIM = 32
NW = 32
COLS_PER_W = 244 * 128
EXTRA_START = NW * COLS_PER_W
REM_START = 999936
ROW_STRIDE = 1000064
TAIL_OFF = DIM * ROW_STRIDE
LIN_SIZE = TAIL_OFF + (VOCAB - REM_START) * DIM
B_PER_W = BATCH // NW


def _make_relayout():
    mesh = plsc.VectorSubcoreMesh(core_axis_name="c", subcore_axis_name="s")

    @functools.partial(
        pl.kernel,
        mesh=mesh,
        out_type=jax.ShapeDtypeStruct((LIN_SIZE,), jnp.float32),
        compiler_params=pltpu.CompilerParams(use_tc_tiling_on_sc=True),
        scratch_types=[
            pltpu.VMEM((COLS_PER_W,), jnp.float32),
            pltpu.VMEM((COLS_PER_W,), jnp.float32),
            pltpu.SemaphoreType.DMA,
            pltpu.SemaphoreType.DMA,
            pltpu.SemaphoreType.DMA,
            pltpu.SemaphoreType.DMA,
            pltpu.SemaphoreType.DMA,
        ],
    )
    def relayout(tableT_hbm, tail_hbm, lin_hbm, b0, b1, sr0, sr1, sw0, sw1,
                 sx):
        cid = lax.axis_index("c")
        sid = lax.axis_index("s")
        wid = cid * 16 + sid
        lo = wid * COLS_PER_W
        bufs = (b0, b1)
        sem_r = (sr0, sr1)
        sem_w = (sw0, sw1)

        def read_of(r):
            return pltpu.make_async_copy(
                tableT_hbm.at[r, pl.ds(lo, COLS_PER_W)],
                bufs[r % 2],
                sem_r[r % 2],
            )

        def write_of(r):
            return pltpu.make_async_copy(
                bufs[r % 2],
                lin_hbm.at[pl.ds(r * ROW_STRIDE + lo, COLS_PER_W)],
                sem_w[r % 2],
            )

        for r in range(DIM):
            if r >= 2:
                write_of(r - 2).wait()
            read_of(r).start()
            read_of(r).wait()
            write_of(r).start()
        write_of(DIM - 2).wait()
        write_of(DIM - 1).wait()

        @pl.when(wid == NW - 1)
        def _():
            for r in range(DIM):
                pltpu.sync_copy(
                    tableT_hbm.at[r, pl.ds(EXTRA_START, 512)],
                    b0.at[pl.ds(0, 512)],
                )
                pltpu.sync_copy(
                    b0.at[pl.ds(0, 512)],
                    lin_hbm.at[pl.ds(r * ROW_STRIDE + EXTRA_START, 512)],
                )

        @pl.when(wid == 0)
        def _():
            pltpu.sync_copy(tail_hbm, b0.at[pl.ds(0, 2048)])
            pltpu.sync_copy(
                b0.at[pl.ds(0, 2048)],
                lin_hbm.at[pl.ds(TAIL_OFF, 2048)],
            )

    return relayout


def _make_gather():
    mesh = plsc.VectorSubcoreMesh(core_axis_name="c", subcore_axis_name="s")
    n_idx_rows = B_PER_W * DIM // 128

    @functools.partial(
        pl.kernel,
        mesh=mesh,
        out_type=jax.ShapeDtypeStruct((BATCH * DIM,), jnp.float32),
        compiler_params=pltpu.CompilerParams(use_tc_tiling_on_sc=False),
        scratch_types=[
            pltpu.VMEM((B_PER_W,), jnp.int32),
            pltpu.VMEM((n_idx_rows, 128), jnp.int32),
            pltpu.VMEM((B_PER_W * DIM,), jnp.float32),
            pltpu.SemaphoreType.DMA,
        ],
    )
    def gather(labels_hbm, lin_hbm, out_hbm, labv, idxb, vbuf, sem):
        cid = lax.axis_index("c")
        sid = lax.axis_index("s")
        wid = cid * 16 + sid
        base = wid * B_PER_W
        iota = lax.iota(jnp.int32, 16)

        pltpu.sync_copy(labels_hbm.at[pl.ds(base, B_PER_W)], labv)

        for j in range(B_PER_W // 16):
            lv = labv[pl.ds(j * 16, 16)]
            mm = (lv - REM_START) >> 31
            tail_idx0 = TAIL_OFF + (lv - REM_START) * DIM
            for d in range(DIM):
                idx_main = lv + d * ROW_STRIDE
                idx_tail = tail_idx0 + d
                idx = (idx_main & mm) | (idx_tail & ~mm)
                idxb[d * 4 + j // 8, pl.ds((j % 8) * 16, 16)] = idx

        copies = []
        for rrow in range(n_idx_rows):
            copies.append(
                pltpu.async_copy(
                    lin_hbm.at[idxb.at[rrow]],
                    vbuf.at[pl.ds(rrow * 128, 128)],
                    sem,
                )
            )
        for c in copies:
            c.wait()

        pltpu.sync_copy(vbuf, out_hbm.at[pl.ds(wid * B_PER_W * DIM, B_PER_W * DIM)])

    return gather


def kernel(labels, table):
    tail = table[REM_START:].reshape(-1)
    lin = _make_relayout()(table.T, tail)
    out_flat = _make_gather()(labels.astype(jnp.int32), lin)
    return (
        out_flat.reshape(NW, DIM, B_PER_W)
        .transpose(0, 2, 1)
        .reshape(BATCH, DIM)
    )
